# baseline (speedup 1.0000x reference)
import dataclasses
import functools
import math

import jax
import jax.numpy as jnp
from jax import lax
from jax.experimental import pallas as pl
from jax.experimental.pallas import tpu as pltpu
from jax.experimental.pallas import tpu_sc as plsc

N_NODES = 10000
N_EDGES = 320000

_ROWT = 1000
_EDGT = 2000

_HIGHEST = jax.lax.Precision.HIGHEST


def _mm(a, b):
    return jax.lax.dot_general(a, b, (((1,), (0,)), ((), ())),
                               preferred_element_type=jnp.float32,
                               precision=_HIGHEST)


def _proj_body(h_ref, h1_ref, sel_ref, w_ref, b_ref,
               q_ref, k_ref, v_ref, s_ref):
    s0 = sel_ref[0:1, 0:1]
    s1 = sel_ref[0:1, 1:2]
    x2 = h_ref[...] * s0 + h1_ref[...] * s1
    xin = jnp.concatenate([h_ref[...], x2], axis=1)
    acc = _mm(xin, w_ref[...]) + b_ref[...]
    q_ref[...] = acc[:, 0:128]
    k_ref[...] = acc[:, 128:256]
    v_ref[...] = acc[:, 256:384]
    s_ref[...] = acc[:, 384:512]


def _proj(h, h1s, sel, wcat, bcat):
    grid = N_NODES // _ROWT
    out = jax.ShapeDtypeStruct((N_NODES, 128), jnp.float32)
    return pl.pallas_call(
        _proj_body,
        grid=(grid,),
        in_specs=[
            pl.BlockSpec((_ROWT, 128), lambda i: (i, 0)),
            pl.BlockSpec((_ROWT, 128), lambda i: (i, 0)),
            pl.BlockSpec((1, 128), lambda i: (0, 0)),
            pl.BlockSpec((256, 512), lambda i: (0, 0)),
            pl.BlockSpec((1, 512), lambda i: (0, 0)),
        ],
        out_specs=[pl.BlockSpec((_ROWT, 128), lambda i: (i, 0))] * 4,
        out_shape=[out, out, out, out],
    )(h, h1s, sel, wcat, bcat)


def _emm_body(a_ref, w_ref, o_ref):
    o_ref[...] = _mm(a_ref[...], w_ref[...])


def _emm(ea, we):
    grid = N_EDGES // _EDGT
    return pl.pallas_call(
        _emm_body,
        grid=(grid,),
        in_specs=[
            pl.BlockSpec((_EDGT, 16), lambda i: (i, 0)),
            pl.BlockSpec((16, 128), lambda i: (0, 0)),
        ],
        out_specs=pl.BlockSpec((_EDGT, 128), lambda i: (i, 0)),
        out_shape=jax.ShapeDtypeStruct((N_EDGES, 128), jnp.float32),
    )(ea, we)


def _post_body(num_ref, den_ref, xr_ref, uw_ref, bmat_ref, y_ref, st_ref):
    den_full = _mm(den_ref[...], bmat_ref[...])
    out = num_ref[...] / (den_full + 1e-16)
    xr = xr_ref[...]
    g = _mm(out, uw_ref[:, 0:1]) + _mm(xr, uw_ref[:, 1:2])
    b = jax.nn.sigmoid(g)
    y = b * xr + (1.0 - b) * out
    y_ref[...] = y

    @pl.when(pl.program_id(0) == 0)
    def _():
        st_ref[...] = jnp.zeros_like(st_ref)

    st_ref[0:1, :] += jnp.sum(y, axis=0, keepdims=True)
    st_ref[1:2, :] += jnp.sum(y * y, axis=0, keepdims=True)


def _post_combine(num, den, xr, uw, bmat):
    grid = N_NODES // _ROWT
    return pl.pallas_call(
        _post_body,
        grid=(grid,),
        in_specs=[
            pl.BlockSpec((_ROWT, 128), lambda i: (i, 0)),
            pl.BlockSpec((_ROWT, 8), lambda i: (i, 0)),
            pl.BlockSpec((_ROWT, 128), lambda i: (i, 0)),
            pl.BlockSpec((128, 2), lambda i: (0, 0)),
            pl.BlockSpec((8, 128), lambda i: (0, 0)),
        ],
        out_specs=[
            pl.BlockSpec((_ROWT, 128), lambda i: (i, 0)),
            pl.BlockSpec((2, 128), lambda i: (0, 0)),
        ],
        out_shape=[
            jax.ShapeDtypeStruct((N_NODES, 128), jnp.float32),
            jax.ShapeDtypeStruct((2, 128), jnp.float32),
        ],
    )(num, den, xr, uw, bmat)


def _bn_body(y_ref, st_ref, gb_ref, fl_ref, o_ref):
    inv_n = 1.0 / float(N_NODES)
    m = st_ref[0:1, :] * inv_n
    ex2 = st_ref[1:2, :] * inv_n
    var = ex2 - m * m
    inv = jax.lax.rsqrt(var + 1e-5)
    y = y_ref[...]
    z = (y - m) * inv * gb_ref[0:1, :] + gb_ref[1:2, :]
    z = jnp.maximum(z, 0.0)
    f = fl_ref[0:1, 0:1]
    o_ref[...] = z * f + y * (1.0 - f)


def _bn_apply(y, stats, gb, flag):
    grid = N_NODES // _ROWT
    return pl.pallas_call(
        _bn_body,
        grid=(grid,),
        in_specs=[
            pl.BlockSpec((_ROWT, 128), lambda i: (i, 0)),
            pl.BlockSpec((2, 128), lambda i: (0, 0)),
            pl.BlockSpec((2, 128), lambda i: (0, 0)),
            pl.BlockSpec((1, 128), lambda i: (0, 0)),
        ],
        out_specs=pl.BlockSpec((_ROWT, 128), lambda i: (i, 0)),
        out_shape=jax.ShapeDtypeStruct((N_NODES, 128), jnp.float32),
    )(y, stats, gb, flag)


_C = 64
_NCHUNK = N_EDGES // _C
_CPT = (_NCHUNK + 15) // 16
_RPT = 624


def _sc_params():
    cp = pltpu.CompilerParams()
    if "needs_layout_passes" in pltpu.CompilerParams.__dataclass_fields__:
        cp = dataclasses.replace(cp, needs_layout_passes=False)
    return cp


def _make_edge_kernel():
    mesh = plsc.VectorSubcoreMesh(core_axis_name="c", subcore_axis_name="s")

    @functools.partial(
        pl.kernel,
        compiler_params=_sc_params(),
        out_type=(jax.ShapeDtypeStruct((N_NODES, 128), jnp.float32),
                  jax.ShapeDtypeStruct((640, 128), jnp.float32)),
        mesh=mesh,
        scratch_types=[
            pltpu.VMEM((_C, 128), jnp.float32),
            pltpu.VMEM((_C, 128), jnp.float32),
            pltpu.VMEM((_C, 128), jnp.float32),
            pltpu.VMEM((_C,), jnp.int32),
            pltpu.VMEM((_C,), jnp.int32),
            pltpu.VMEM((_C,), jnp.int32),
            pltpu.VMEM((_C, 128), jnp.float32),
            pltpu.VMEM((16,), jnp.float32),
            pltpu.VMEM((8, 128), jnp.float32),
            pltpu.VMEM_SHARED((N_NODES, 128), jnp.float32),
            pltpu.VMEM_SHARED((640, 128), jnp.float32),
        ],
    )
    def ek(q_hbm, k_hbm, v_hbm, e_hbm, src_hbm, dst_hbm, m_hbm,
           num_hbm, den_hbm,
           qbuf, kbuf, ebuf, sbuf, dbuf, dbuf16, exbuf, mbuf, zbuf,
           num_sh, den_sh):
        sid = lax.axis_index("s")
        zero16f = jnp.zeros((16,), jnp.float32)
        pltpu.sync_copy(m_hbm, mbuf)

        @pl.loop(0, 8)
        def _(r):
            for c in range(8):
                zbuf[r, pl.ds(c * 16, 16)] = zero16f

        @pl.loop(0, _C)
        def _(r):
            for c in range(8):
                exbuf[r, pl.ds(c * 16, 16)] = zero16f

        row0 = sid * _RPT
        row0d = sid * 40

        @pl.loop(0, 78)
        def _(j):
            pltpu.sync_copy(zbuf, num_sh.at[pl.ds(row0 + j * 8, 8)])

        @pl.loop(0, 5)
        def _(j):
            pltpu.sync_copy(zbuf, den_sh.at[pl.ds(row0d + j * 8, 8)])

        @pl.when(sid == 15)
        def _():
            pltpu.sync_copy(zbuf, num_sh.at[pl.ds(row0 + 624, 8)])
            pltpu.sync_copy(zbuf, num_sh.at[pl.ds(row0 + 632, 8)])

        plsc.subcore_barrier()
        mv = mbuf[:]
        ma = mv[0]
        mb = mv[1]

        @pl.loop(0, _CPT)
        def _(j):
            cidx = sid + j * 16

            @pl.when(cidx < _NCHUNK)
            def _():
                base = cidx * _C
                pltpu.sync_copy(src_hbm.at[pl.ds(base, _C)], sbuf)
                pltpu.sync_copy(dst_hbm.at[pl.ds(base, _C)], dbuf)
                pltpu.sync_copy(q_hbm.at[dbuf], qbuf)
                pltpu.sync_copy(k_hbm.at[sbuf], kbuf)
                pltpu.sync_copy(e_hbm.at[pl.ds(base, _C)], ebuf)

                @pl.loop(0, _C // 16)
                def _(g):
                    rows = lax.iota(jnp.int32, 16) + g * 16
                    sl = pl.ds(g * 16, 16)
                    dstv = dbuf[sl]
                    dbuf16[sl] = jnp.right_shift(dstv, 4)
                    lane = jnp.bitwise_and(dstv, 15) * 8
                    ss = []
                    for h in range(4):
                        acc0 = jnp.zeros((16,), jnp.float32)
                        acc1 = jnp.zeros((16,), jnp.float32)
                        for t in range(32):
                            col = jnp.full((16,), h * 32 + t, jnp.int32)
                            qv = plsc.load_gather(qbuf, [rows, col])
                            kv = plsc.load_gather(kbuf, [rows, col])
                            ev = plsc.load_gather(ebuf, [rows, col])
                            if t % 2 == 0:
                                acc0 = acc0 + qv * (kv + ev)
                            else:
                                acc1 = acc1 + qv * (kv + ev)
                        ss.append(acc0 + acc1)
                    stot = (ss[0] + ss[1]) + (ss[2] + ss[3])
                    for h in range(4):
                        exh = jnp.exp(ss[h] * ma + stot * mb)
                        plsc.store_scatter(exbuf, [rows, lane + h], exh)

                pltpu.sync_copy(v_hbm.at[sbuf], qbuf)

                @pl.loop(0, _C // 16)
                def _(g):
                    rows = lax.iota(jnp.int32, 16) + g * 16
                    sl = pl.ds(g * 16, 16)
                    lane = jnp.bitwise_and(dbuf[sl], 15) * 8
                    for h in range(4):
                        exh = plsc.load_gather(exbuf, [rows, lane + h])
                        for t in range(32):
                            col = jnp.full((16,), h * 32 + t, jnp.int32)
                            vv = plsc.load_gather(qbuf, [rows, col])
                            ev = plsc.load_gather(ebuf, [rows, col])
                            plsc.store_scatter(
                                qbuf, [rows, col], (vv + ev) * exh)

                pltpu.sync_copy(qbuf, num_sh.at[dbuf], add=True)
                pltpu.sync_copy(exbuf, den_sh.at[dbuf16], add=True)

                @pl.loop(0, _C // 16)
                def _(g):
                    rows = lax.iota(jnp.int32, 16) + g * 16
                    lane = jnp.bitwise_and(dbuf[pl.ds(g * 16, 16)], 15) * 8
                    for h in range(4):
                        plsc.store_scatter(exbuf, [rows, lane + h], zero16f)

        plsc.subcore_barrier()

        @pl.loop(0, 78)
        def _(j):
            pltpu.sync_copy(num_sh.at[pl.ds(row0 + j * 8, 8)],
                            num_hbm.at[pl.ds(row0 + j * 8, 8)])

        @pl.loop(0, 5)
        def _(j):
            pltpu.sync_copy(den_sh.at[pl.ds(row0d + j * 8, 8)],
                            den_hbm.at[pl.ds(row0d + j * 8, 8)])

        @pl.when(sid == 15)
        def _():
            pltpu.sync_copy(num_sh.at[pl.ds(row0 + 624, 8)],
                            num_hbm.at[pl.ds(row0 + 624, 8)])
            pltpu.sync_copy(num_sh.at[pl.ds(row0 + 632, 8)],
                            num_hbm.at[pl.ds(row0 + 632, 8)])

    return ek


def _stack_params(P):
    ws, bs, wes, uws, mvecs, gbs, flags = [], [], [], [], [], [], []
    specs = [('enc0', False, True), ('enc1', False, True),
             ('dec0', True, True), ('dec1', True, False)]
    bns = [('bn0_g', 'bn0_b'), ('bn1_g', 'bn1_b'), ('bn2_g', 'bn2_b'), None]
    for (name, wide, four_heads), bn in zip(specs, bns):
        p = P[name]
        wcat = jnp.concatenate([p['Wq'], p['Wk'], p['Wv'], p['Ws']], axis=1)
        if not wide:
            wcat = jnp.concatenate(
                [wcat, jnp.zeros((128, 512), jnp.float32)], axis=0)
        ws.append(wcat)
        bs.append(jnp.concatenate(
            [p['bq'], p['bk'], p['bv'], p['bs']]).reshape(1, 512))
        wes.append(p['We'])
        wb = p['Wb'][:, 0]
        uws.append(jnp.stack(
            [wb[0:128] + wb[256:384], wb[128:256] - wb[256:384]], axis=1))
        mv = jnp.zeros((16,), jnp.float32)
        if four_heads:
            mv = mv.at[0].set(1.0 / math.sqrt(32.0))
        else:
            mv = mv.at[1].set(1.0 / math.sqrt(128.0))
        mvecs.append(mv)
        if bn is None:
            gbs.append(jnp.zeros((2, 128), jnp.float32))
            flags.append(jnp.zeros((1, 128), jnp.float32))
        else:
            gbs.append(jnp.stack([P[bn[0]], P[bn[1]]]))
            flags.append(jnp.ones((1, 128), jnp.float32))
    sels = [jnp.zeros((1, 128), jnp.float32),
            jnp.zeros((1, 128), jnp.float32),
            jnp.zeros((1, 128), jnp.float32).at[0, 0].set(1.0),
            jnp.zeros((1, 128), jnp.float32).at[0, 1].set(1.0)]
    keep_h1 = jnp.array([1.0, 0.0, 0.0, 0.0], jnp.float32)
    return (jnp.stack(ws), jnp.stack(bs), jnp.stack(wes), jnp.stack(uws),
            jnp.stack(mvecs), jnp.stack(gbs), jnp.stack(flags),
            jnp.stack(sels), keep_h1)


def kernel(x, edge_index, edge_attr, params):
    src = edge_index[0]
    dst = edge_index[1]
    xs = _stack_params(params)
    bmat = (jnp.arange(128)[None, :] // 32 == jnp.arange(8)[:, None]
            ).astype(jnp.float32)
    ek = _make_edge_kernel()

    def body(carry, per):
        h, h1s = carry
        wcat, bcat, we, uw, mvec, gb, flag, sel, keep = per
        q, k, v, xr = _proj(h, h1s, sel, wcat, bcat)
        e_all = _emm(edge_attr, we)
        num, denp = ek(q, k, v, e_all, src, dst, mvec)
        den = denp.reshape(640 * 16, 8)[:N_NODES]
        y, stats = _post_combine(num, den, xr, uw, bmat)
        z = _bn_apply(y, stats, gb, flag)
        h1s_new = h1s * (1.0 - keep) + z * keep
        return (z, h1s_new), None

    (out, _), _ = jax.lax.scan(
        body, (x, jnp.zeros((N_NODES, 128), jnp.float32)), xs)
    return out


# speedup vs baseline: 1.0612x; 1.0612x over previous
import dataclasses
import functools
import math

import jax
import jax.numpy as jnp
from jax import lax
from jax.experimental import pallas as pl
from jax.experimental.pallas import tpu as pltpu
from jax.experimental.pallas import tpu_sc as plsc

N_NODES = 10000
N_EDGES = 320000

_ROWT = 1000
_EDGT = 2000

_HIGHEST = jax.lax.Precision.HIGHEST


def _mm(a, b):
    return jax.lax.dot_general(a, b, (((1,), (0,)), ((), ())),
                               preferred_element_type=jnp.float32,
                               precision=_HIGHEST)


def _proj_body(h_ref, h1_ref, sel_ref, w_ref, b_ref,
               q_ref, k_ref, v_ref, s_ref):
    s0 = sel_ref[0:1, 0:1]
    s1 = sel_ref[0:1, 1:2]
    x2 = h_ref[...] * s0 + h1_ref[...] * s1
    xin = jnp.concatenate([h_ref[...], x2], axis=1)
    acc = _mm(xin, w_ref[...]) + b_ref[...]
    q_ref[...] = acc[:, 0:128]
    k_ref[...] = acc[:, 128:256]
    v_ref[...] = acc[:, 256:384]
    s_ref[...] = acc[:, 384:512]


def _proj(h, h1s, sel, wcat, bcat):
    grid = N_NODES // _ROWT
    out = jax.ShapeDtypeStruct((N_NODES, 128), jnp.float32)
    return pl.pallas_call(
        _proj_body,
        grid=(grid,),
        in_specs=[
            pl.BlockSpec((_ROWT, 128), lambda i: (i, 0)),
            pl.BlockSpec((_ROWT, 128), lambda i: (i, 0)),
            pl.BlockSpec((1, 128), lambda i: (0, 0)),
            pl.BlockSpec((256, 512), lambda i: (0, 0)),
            pl.BlockSpec((1, 512), lambda i: (0, 0)),
        ],
        out_specs=[pl.BlockSpec((_ROWT, 128), lambda i: (i, 0))] * 4,
        out_shape=[out, out, out, out],
    )(h, h1s, sel, wcat, bcat)


def _emm_body(a_ref, w_ref, o_ref):
    o_ref[...] = _mm(a_ref[...], w_ref[...])


def _emm(ea, we):
    grid = N_EDGES // _EDGT
    return pl.pallas_call(
        _emm_body,
        grid=(grid,),
        in_specs=[
            pl.BlockSpec((_EDGT, 16), lambda i: (i, 0)),
            pl.BlockSpec((16, 128), lambda i: (0, 0)),
        ],
        out_specs=pl.BlockSpec((_EDGT, 128), lambda i: (i, 0)),
        out_shape=jax.ShapeDtypeStruct((N_EDGES, 128), jnp.float32),
    )(ea, we)


def _post_body(num_ref, den_ref, xr_ref, uw_ref, bmat_ref, y_ref, st_ref):
    den_full = _mm(den_ref[...], bmat_ref[...])
    out = num_ref[...] / (den_full + 1e-16)
    xr = xr_ref[...]
    g = _mm(out, uw_ref[:, 0:1]) + _mm(xr, uw_ref[:, 1:2])
    b = jax.nn.sigmoid(g)
    y = b * xr + (1.0 - b) * out
    y_ref[...] = y

    @pl.when(pl.program_id(0) == 0)
    def _():
        st_ref[...] = jnp.zeros_like(st_ref)

    st_ref[0:1, :] += jnp.sum(y, axis=0, keepdims=True)
    st_ref[1:2, :] += jnp.sum(y * y, axis=0, keepdims=True)


def _post_combine(num, den, xr, uw, bmat):
    grid = N_NODES // _ROWT
    return pl.pallas_call(
        _post_body,
        grid=(grid,),
        in_specs=[
            pl.BlockSpec((_ROWT, 128), lambda i: (i, 0)),
            pl.BlockSpec((_ROWT, 8), lambda i: (i, 0)),
            pl.BlockSpec((_ROWT, 128), lambda i: (i, 0)),
            pl.BlockSpec((128, 2), lambda i: (0, 0)),
            pl.BlockSpec((8, 128), lambda i: (0, 0)),
        ],
        out_specs=[
            pl.BlockSpec((_ROWT, 128), lambda i: (i, 0)),
            pl.BlockSpec((2, 128), lambda i: (0, 0)),
        ],
        out_shape=[
            jax.ShapeDtypeStruct((N_NODES, 128), jnp.float32),
            jax.ShapeDtypeStruct((2, 128), jnp.float32),
        ],
    )(num, den, xr, uw, bmat)


def _bn_body(y_ref, st_ref, gb_ref, fl_ref, o_ref):
    inv_n = 1.0 / float(N_NODES)
    m = st_ref[0:1, :] * inv_n
    ex2 = st_ref[1:2, :] * inv_n
    var = ex2 - m * m
    inv = jax.lax.rsqrt(var + 1e-5)
    y = y_ref[...]
    z = (y - m) * inv * gb_ref[0:1, :] + gb_ref[1:2, :]
    z = jnp.maximum(z, 0.0)
    f = fl_ref[0:1, 0:1]
    o_ref[...] = z * f + y * (1.0 - f)


def _bn_apply(y, stats, gb, flag):
    grid = N_NODES // _ROWT
    return pl.pallas_call(
        _bn_body,
        grid=(grid,),
        in_specs=[
            pl.BlockSpec((_ROWT, 128), lambda i: (i, 0)),
            pl.BlockSpec((2, 128), lambda i: (0, 0)),
            pl.BlockSpec((2, 128), lambda i: (0, 0)),
            pl.BlockSpec((1, 128), lambda i: (0, 0)),
        ],
        out_specs=pl.BlockSpec((_ROWT, 128), lambda i: (i, 0)),
        out_shape=jax.ShapeDtypeStruct((N_NODES, 128), jnp.float32),
    )(y, stats, gb, flag)


_C = 64
_NCHUNK = N_EDGES // _C
_CPT = (_NCHUNK + 15) // 16
_RPT = 624


def _sc_params():
    cp = pltpu.CompilerParams()
    if "needs_layout_passes" in pltpu.CompilerParams.__dataclass_fields__:
        cp = dataclasses.replace(cp, needs_layout_passes=False)
    return cp


def _make_edge_kernel():
    mesh = plsc.VectorSubcoreMesh(core_axis_name="c", subcore_axis_name="s")

    @functools.partial(
        pl.kernel,
        compiler_params=_sc_params(),
        out_type=(jax.ShapeDtypeStruct((N_NODES, 128), jnp.float32),
                  jax.ShapeDtypeStruct((640, 128), jnp.float32)),
        mesh=mesh,
        scratch_types=[
            pltpu.VMEM((_C, 128), jnp.float32),
            pltpu.VMEM((_C, 128), jnp.float32),
            pltpu.VMEM((_C, 128), jnp.float32),
            pltpu.VMEM((_C,), jnp.int32),
            pltpu.VMEM((_C,), jnp.int32),
            pltpu.VMEM((_C,), jnp.int32),
            pltpu.VMEM((_C, 128), jnp.float32),
            pltpu.VMEM((16,), jnp.float32),
            pltpu.VMEM((8, 128), jnp.float32),
            pltpu.VMEM_SHARED((N_NODES, 128), jnp.float32),
            pltpu.VMEM_SHARED((640, 128), jnp.float32),
            pltpu.SemaphoreType.DMA,
            pltpu.SemaphoreType.DMA,
            pltpu.SemaphoreType.DMA,
        ],
    )
    def ek(q_hbm, k_hbm, v_hbm, e_hbm, src_hbm, dst_hbm, m_hbm,
           num_hbm, den_hbm,
           qbuf, kbuf, ebuf, sbuf, dbuf, dbuf16, exbuf, mbuf, zbuf,
           num_sh, den_sh, sem_i, sem_g, sem_s):
        sid = lax.axis_index("s")
        zero16f = jnp.zeros((16,), jnp.float32)
        pltpu.sync_copy(m_hbm, mbuf)

        @pl.loop(0, 8)
        def _(r):
            for c in range(8):
                zbuf[r, pl.ds(c * 16, 16)] = zero16f

        @pl.loop(0, _C)
        def _(r):
            for c in range(8):
                exbuf[r, pl.ds(c * 16, 16)] = zero16f

        row0 = sid * _RPT
        row0d = sid * 40

        @pl.loop(0, 78)
        def _(j):
            pltpu.sync_copy(zbuf, num_sh.at[pl.ds(row0 + j * 8, 8)])

        @pl.loop(0, 5)
        def _(j):
            pltpu.sync_copy(zbuf, den_sh.at[pl.ds(row0d + j * 8, 8)])

        @pl.when(sid == 15)
        def _():
            pltpu.sync_copy(zbuf, num_sh.at[pl.ds(row0 + 624, 8)])
            pltpu.sync_copy(zbuf, num_sh.at[pl.ds(row0 + 632, 8)])

        plsc.subcore_barrier()
        mv = mbuf[:]
        ma = mv[0]
        mb = mv[1]

        @pl.loop(0, _CPT)
        def _(j):
            cidx = sid + j * 16

            @pl.when(cidx < _NCHUNK)
            def _():
                base = cidx * _C
                ci1 = pltpu.async_copy(src_hbm.at[pl.ds(base, _C)], sbuf,
                                       sem_i)
                ci2 = pltpu.async_copy(dst_hbm.at[pl.ds(base, _C)], dbuf,
                                       sem_i)
                ce = pltpu.async_copy(e_hbm.at[pl.ds(base, _C)], ebuf, sem_g)
                ci1.wait()
                ci2.wait()
                cq = pltpu.async_copy(q_hbm.at[dbuf], qbuf, sem_g)
                ck = pltpu.async_copy(k_hbm.at[sbuf], kbuf, sem_g)
                ce.wait()
                cq.wait()
                ck.wait()

                @pl.loop(0, _C // 16)
                def _(g):
                    rows = lax.iota(jnp.int32, 16) + g * 16
                    sl = pl.ds(g * 16, 16)
                    dstv = dbuf[sl]
                    dbuf16[sl] = jnp.right_shift(dstv, 4)
                    lane = jnp.bitwise_and(dstv, 15) * 8
                    ss = []
                    for h in range(4):
                        acc0 = jnp.zeros((16,), jnp.float32)
                        acc1 = jnp.zeros((16,), jnp.float32)
                        for t in range(32):
                            col = jnp.full((16,), h * 32 + t, jnp.int32)
                            qv = plsc.load_gather(qbuf, [rows, col])
                            kv = plsc.load_gather(kbuf, [rows, col])
                            ev = plsc.load_gather(ebuf, [rows, col])
                            if t % 2 == 0:
                                acc0 = acc0 + qv * (kv + ev)
                            else:
                                acc1 = acc1 + qv * (kv + ev)
                        ss.append(acc0 + acc1)
                    stot = (ss[0] + ss[1]) + (ss[2] + ss[3])
                    for h in range(4):
                        exh = jnp.exp(ss[h] * ma + stot * mb)
                        plsc.store_scatter(exbuf, [rows, lane + h], exh)

                pltpu.async_copy(v_hbm.at[sbuf], qbuf, sem_g).wait()

                @pl.loop(0, _C // 16)
                def _(g):
                    rows = lax.iota(jnp.int32, 16) + g * 16
                    sl = pl.ds(g * 16, 16)
                    lane = jnp.bitwise_and(dbuf[sl], 15) * 8
                    for h in range(4):
                        exh = plsc.load_gather(exbuf, [rows, lane + h])
                        for t in range(32):
                            col = jnp.full((16,), h * 32 + t, jnp.int32)
                            vv = plsc.load_gather(qbuf, [rows, col])
                            ev = plsc.load_gather(ebuf, [rows, col])
                            plsc.store_scatter(
                                qbuf, [rows, col], (vv + ev) * exh)

                cs1 = pltpu.async_copy(qbuf, num_sh.at[dbuf], sem_s,
                                       add=True)
                cs2 = pltpu.async_copy(exbuf, den_sh.at[dbuf16], sem_s,
                                       add=True)
                cs1.wait()
                cs2.wait()

                @pl.loop(0, _C // 16)
                def _(g):
                    rows = lax.iota(jnp.int32, 16) + g * 16
                    lane = jnp.bitwise_and(dbuf[pl.ds(g * 16, 16)], 15) * 8
                    for h in range(4):
                        plsc.store_scatter(exbuf, [rows, lane + h], zero16f)

        plsc.subcore_barrier()

        @pl.loop(0, 78)
        def _(j):
            pltpu.sync_copy(num_sh.at[pl.ds(row0 + j * 8, 8)],
                            num_hbm.at[pl.ds(row0 + j * 8, 8)])

        @pl.loop(0, 5)
        def _(j):
            pltpu.sync_copy(den_sh.at[pl.ds(row0d + j * 8, 8)],
                            den_hbm.at[pl.ds(row0d + j * 8, 8)])

        @pl.when(sid == 15)
        def _():
            pltpu.sync_copy(num_sh.at[pl.ds(row0 + 624, 8)],
                            num_hbm.at[pl.ds(row0 + 624, 8)])
            pltpu.sync_copy(num_sh.at[pl.ds(row0 + 632, 8)],
                            num_hbm.at[pl.ds(row0 + 632, 8)])

    return ek


def _stack_params(P):
    ws, bs, wes, uws, mvecs, gbs, flags = [], [], [], [], [], [], []
    specs = [('enc0', False, True), ('enc1', False, True),
             ('dec0', True, True), ('dec1', True, False)]
    bns = [('bn0_g', 'bn0_b'), ('bn1_g', 'bn1_b'), ('bn2_g', 'bn2_b'), None]
    for (name, wide, four_heads), bn in zip(specs, bns):
        p = P[name]
        wcat = jnp.concatenate([p['Wq'], p['Wk'], p['Wv'], p['Ws']], axis=1)
        if not wide:
            wcat = jnp.concatenate(
                [wcat, jnp.zeros((128, 512), jnp.float32)], axis=0)
        ws.append(wcat)
        bs.append(jnp.concatenate(
            [p['bq'], p['bk'], p['bv'], p['bs']]).reshape(1, 512))
        wes.append(p['We'])
        wb = p['Wb'][:, 0]
        uws.append(jnp.stack(
            [wb[0:128] + wb[256:384], wb[128:256] - wb[256:384]], axis=1))
        mv = jnp.zeros((16,), jnp.float32)
        if four_heads:
            mv = mv.at[0].set(1.0 / math.sqrt(32.0))
        else:
            mv = mv.at[1].set(1.0 / math.sqrt(128.0))
        mvecs.append(mv)
        if bn is None:
            gbs.append(jnp.zeros((2, 128), jnp.float32))
            flags.append(jnp.zeros((1, 128), jnp.float32))
        else:
            gbs.append(jnp.stack([P[bn[0]], P[bn[1]]]))
            flags.append(jnp.ones((1, 128), jnp.float32))
    sels = [jnp.zeros((1, 128), jnp.float32),
            jnp.zeros((1, 128), jnp.float32),
            jnp.zeros((1, 128), jnp.float32).at[0, 0].set(1.0),
            jnp.zeros((1, 128), jnp.float32).at[0, 1].set(1.0)]
    keep_h1 = jnp.array([1.0, 0.0, 0.0, 0.0], jnp.float32)
    return (jnp.stack(ws), jnp.stack(bs), jnp.stack(wes), jnp.stack(uws),
            jnp.stack(mvecs), jnp.stack(gbs), jnp.stack(flags),
            jnp.stack(sels), keep_h1)


def kernel(x, edge_index, edge_attr, params):
    src = edge_index[0]
    dst = edge_index[1]
    xs = _stack_params(params)
    bmat = (jnp.arange(128)[None, :] // 32 == jnp.arange(8)[:, None]
            ).astype(jnp.float32)
    ek = _make_edge_kernel()

    def body(carry, per):
        h, h1s = carry
        wcat, bcat, we, uw, mvec, gb, flag, sel, keep = per
        q, k, v, xr = _proj(h, h1s, sel, wcat, bcat)
        e_all = _emm(edge_attr, we)
        num, denp = ek(q, k, v, e_all, src, dst, mvec)
        den = denp.reshape(640 * 16, 8)[:N_NODES]
        y, stats = _post_combine(num, den, xr, uw, bmat)
        z = _bn_apply(y, stats, gb, flag)
        h1s_new = h1s * (1.0 - keep) + z * keep
        return (z, h1s_new), None

    (out, _), _ = jax.lax.scan(
        body, (x, jnp.zeros((N_NODES, 128), jnp.float32)), xs)
    return out


# speedup vs baseline: 2.7161x; 2.5595x over previous
import dataclasses
import functools
import math

import jax
import jax.numpy as jnp
from jax import lax
from jax.experimental import pallas as pl
from jax.experimental.pallas import tpu as pltpu
from jax.experimental.pallas import tpu_sc as plsc

N_NODES = 10000
N_EDGES = 320000

_ROWT = 1000
_EDGT = 2000

_HIGHEST = jax.lax.Precision.HIGHEST


def _mm(a, b):
    return jax.lax.dot_general(a, b, (((1,), (0,)), ((), ())),
                               preferred_element_type=jnp.float32,
                               precision=_HIGHEST)


def _proj_body(h_ref, h1_ref, sel_ref, w_ref, b_ref,
               q_ref, k_ref, v_ref, s_ref):
    s0 = sel_ref[0:1, 0:1]
    s1 = sel_ref[0:1, 1:2]
    x2 = h_ref[...] * s0 + h1_ref[...] * s1
    xin = jnp.concatenate([h_ref[...], x2], axis=1)
    acc = _mm(xin, w_ref[...]) + b_ref[...]
    q_ref[...] = acc[:, 0:128]
    k_ref[...] = acc[:, 128:256]
    v_ref[...] = acc[:, 256:384]
    s_ref[...] = acc[:, 384:512]


def _proj(h, h1s, sel, wcat, bcat):
    grid = N_NODES // _ROWT
    out = jax.ShapeDtypeStruct((N_NODES, 128), jnp.float32)
    return pl.pallas_call(
        _proj_body,
        grid=(grid,),
        in_specs=[
            pl.BlockSpec((_ROWT, 128), lambda i: (i, 0)),
            pl.BlockSpec((_ROWT, 128), lambda i: (i, 0)),
            pl.BlockSpec((1, 128), lambda i: (0, 0)),
            pl.BlockSpec((256, 512), lambda i: (0, 0)),
            pl.BlockSpec((1, 512), lambda i: (0, 0)),
        ],
        out_specs=[pl.BlockSpec((_ROWT, 128), lambda i: (i, 0))] * 4,
        out_shape=[out, out, out, out],
    )(h, h1s, sel, wcat, bcat)


def _emm_body(a_ref, w_ref, o_ref):
    o_ref[...] = _mm(a_ref[...], w_ref[...])


def _emm(ea, we):
    grid = N_EDGES // _EDGT
    return pl.pallas_call(
        _emm_body,
        grid=(grid,),
        in_specs=[
            pl.BlockSpec((_EDGT, 16), lambda i: (i, 0)),
            pl.BlockSpec((16, 128), lambda i: (0, 0)),
        ],
        out_specs=pl.BlockSpec((_EDGT, 128), lambda i: (i, 0)),
        out_shape=jax.ShapeDtypeStruct((N_EDGES, 128), jnp.float32),
    )(ea, we)


def _post_body(num_ref, den_ref, xr_ref, uw_ref, bmat_ref, y_ref, st_ref):
    den_full = _mm(den_ref[...], bmat_ref[...])
    out = num_ref[...] / (den_full + 1e-16)
    xr = xr_ref[...]
    g = _mm(out, uw_ref[:, 0:1]) + _mm(xr, uw_ref[:, 1:2])
    b = jax.nn.sigmoid(g)
    y = b * xr + (1.0 - b) * out
    y_ref[...] = y

    @pl.when(pl.program_id(0) == 0)
    def _():
        st_ref[...] = jnp.zeros_like(st_ref)

    st_ref[0:1, :] += jnp.sum(y, axis=0, keepdims=True)
    st_ref[1:2, :] += jnp.sum(y * y, axis=0, keepdims=True)


def _post_combine(num, den, xr, uw, bmat):
    grid = N_NODES // _ROWT
    return pl.pallas_call(
        _post_body,
        grid=(grid,),
        in_specs=[
            pl.BlockSpec((_ROWT, 128), lambda i: (i, 0)),
            pl.BlockSpec((_ROWT, 8), lambda i: (i, 0)),
            pl.BlockSpec((_ROWT, 128), lambda i: (i, 0)),
            pl.BlockSpec((128, 2), lambda i: (0, 0)),
            pl.BlockSpec((8, 128), lambda i: (0, 0)),
        ],
        out_specs=[
            pl.BlockSpec((_ROWT, 128), lambda i: (i, 0)),
            pl.BlockSpec((2, 128), lambda i: (0, 0)),
        ],
        out_shape=[
            jax.ShapeDtypeStruct((N_NODES, 128), jnp.float32),
            jax.ShapeDtypeStruct((2, 128), jnp.float32),
        ],
    )(num, den, xr, uw, bmat)


def _bn_body(y_ref, st_ref, gb_ref, fl_ref, o_ref):
    inv_n = 1.0 / float(N_NODES)
    m = st_ref[0:1, :] * inv_n
    ex2 = st_ref[1:2, :] * inv_n
    var = ex2 - m * m
    inv = jax.lax.rsqrt(var + 1e-5)
    y = y_ref[...]
    z = (y - m) * inv * gb_ref[0:1, :] + gb_ref[1:2, :]
    z = jnp.maximum(z, 0.0)
    f = fl_ref[0:1, 0:1]
    o_ref[...] = z * f + y * (1.0 - f)


def _bn_apply(y, stats, gb, flag):
    grid = N_NODES // _ROWT
    return pl.pallas_call(
        _bn_body,
        grid=(grid,),
        in_specs=[
            pl.BlockSpec((_ROWT, 128), lambda i: (i, 0)),
            pl.BlockSpec((2, 128), lambda i: (0, 0)),
            pl.BlockSpec((2, 128), lambda i: (0, 0)),
            pl.BlockSpec((1, 128), lambda i: (0, 0)),
        ],
        out_specs=pl.BlockSpec((_ROWT, 128), lambda i: (i, 0)),
        out_shape=jax.ShapeDtypeStruct((N_NODES, 128), jnp.float32),
    )(y, stats, gb, flag)


_C = 64
_NCHUNK = N_EDGES // _C
_CPT = (_NCHUNK + 15) // 16
_RPT = 624


def _sc_params():
    cp = pltpu.CompilerParams()
    if "needs_layout_passes" in pltpu.CompilerParams.__dataclass_fields__:
        cp = dataclasses.replace(cp, needs_layout_passes=False)
    return cp


def _make_edge_kernel():
    mesh = plsc.VectorSubcoreMesh(core_axis_name="c", subcore_axis_name="s")

    @functools.partial(
        pl.kernel,
        compiler_params=_sc_params(),
        out_type=(jax.ShapeDtypeStruct((N_NODES, 128), jnp.float32),
                  jax.ShapeDtypeStruct((640, 128), jnp.float32)),
        mesh=mesh,
        scratch_types=[
            pltpu.VMEM((_C, 128), jnp.float32),
            pltpu.VMEM((_C, 128), jnp.float32),
            pltpu.VMEM((_C, 128), jnp.float32),
            pltpu.VMEM((_C,), jnp.int32),
            pltpu.VMEM((_C,), jnp.int32),
            pltpu.VMEM((_C,), jnp.int32),
            pltpu.VMEM((_C, 128), jnp.float32),
            pltpu.VMEM((16,), jnp.float32),
            pltpu.VMEM((8, 128), jnp.float32),
            pltpu.VMEM_SHARED((N_NODES, 128), jnp.float32),
            pltpu.VMEM_SHARED((640, 128), jnp.float32),
            pltpu.SemaphoreType.DMA,
            pltpu.SemaphoreType.DMA,
            pltpu.SemaphoreType.DMA,
        ],
    )
    def ek(q_hbm, k_hbm, v_hbm, e_hbm, src_hbm, dst_hbm, m_hbm,
           num_hbm, den_hbm,
           qbuf, kbuf, ebuf, sbuf, dbuf, dbuf16, exbuf, mbuf, zbuf,
           num_sh, den_sh, sem_i, sem_g, sem_s):
        sid = lax.axis_index("s")
        zero16f = jnp.zeros((16,), jnp.float32)
        pltpu.sync_copy(m_hbm, mbuf)

        @pl.loop(0, 8)
        def _(r):
            for c in range(8):
                zbuf[r, pl.ds(c * 16, 16)] = zero16f

        @pl.loop(0, _C)
        def _(r):
            for c in range(8):
                exbuf[r, pl.ds(c * 16, 16)] = zero16f

        row0 = sid * _RPT
        row0d = sid * 40

        @pl.loop(0, 78)
        def _(j):
            pltpu.sync_copy(zbuf, num_sh.at[pl.ds(row0 + j * 8, 8)])

        @pl.loop(0, 5)
        def _(j):
            pltpu.sync_copy(zbuf, den_sh.at[pl.ds(row0d + j * 8, 8)])

        @pl.when(sid == 15)
        def _():
            pltpu.sync_copy(zbuf, num_sh.at[pl.ds(row0 + 624, 8)])
            pltpu.sync_copy(zbuf, num_sh.at[pl.ds(row0 + 632, 8)])

        plsc.subcore_barrier()
        mv = mbuf[:]
        ma = mv[0]
        mb = mv[1]

        @pl.loop(0, _CPT)
        def _(j):
            cidx = sid + j * 16

            @pl.when(cidx < _NCHUNK)
            def _():
                base = cidx * _C
                ci1 = pltpu.async_copy(src_hbm.at[pl.ds(base, _C)], sbuf,
                                       sem_i)
                ci2 = pltpu.async_copy(dst_hbm.at[pl.ds(base, _C)], dbuf,
                                       sem_i)
                ce = pltpu.async_copy(e_hbm.at[pl.ds(base, _C)], ebuf, sem_g)
                ci1.wait()
                ci2.wait()
                cq = pltpu.async_copy(q_hbm.at[dbuf], qbuf, sem_g)
                ck = pltpu.async_copy(k_hbm.at[sbuf], kbuf, sem_g)
                ce.wait()
                cq.wait()
                ck.wait()

                @pl.loop(0, _C // 16)
                def _(g):
                    iota = lax.iota(jnp.int32, 16)
                    rows = iota + g * 16
                    sl = pl.ds(g * 16, 16)
                    dstv = dbuf[sl]
                    dbuf16[sl] = jnp.right_shift(dstv, 4)
                    lane = jnp.bitwise_and(dstv, 15) * 8
                    ss = []
                    for h in range(4):
                        acc0 = jnp.zeros((16,), jnp.float32)
                        acc1 = jnp.zeros((16,), jnp.float32)
                        for w in range(2):
                            base = h * 32 + 16 * w
                            for t in range(16):
                                col = jnp.bitwise_and(iota + t, 15) + base
                                qv = plsc.load_gather(qbuf, [rows, col])
                                kv = plsc.load_gather(kbuf, [rows, col])
                                ev = plsc.load_gather(ebuf, [rows, col])
                                if t % 2 == 0:
                                    acc0 = acc0 + qv * (kv + ev)
                                else:
                                    acc1 = acc1 + qv * (kv + ev)
                        ss.append(acc0 + acc1)
                    stot = (ss[0] + ss[1]) + (ss[2] + ss[3])
                    for h in range(4):
                        exh = jnp.exp(ss[h] * ma + stot * mb)
                        plsc.store_scatter(exbuf, [rows, lane + h], exh)

                pltpu.async_copy(v_hbm.at[sbuf], qbuf, sem_g).wait()

                @pl.loop(0, _C // 16)
                def _(g):
                    iota = lax.iota(jnp.int32, 16)
                    rows = iota + g * 16
                    sl = pl.ds(g * 16, 16)
                    lane = jnp.bitwise_and(dbuf[sl], 15) * 8
                    for h in range(4):
                        exh = plsc.load_gather(exbuf, [rows, lane + h])
                        for w in range(2):
                            base = h * 32 + 16 * w
                            for t in range(16):
                                col = jnp.bitwise_and(iota + t, 15) + base
                                vv = plsc.load_gather(qbuf, [rows, col])
                                ev = plsc.load_gather(ebuf, [rows, col])
                                plsc.store_scatter(
                                    qbuf, [rows, col], (vv + ev) * exh)

                cs1 = pltpu.async_copy(qbuf, num_sh.at[dbuf], sem_s,
                                       add=True)
                cs2 = pltpu.async_copy(exbuf, den_sh.at[dbuf16], sem_s,
                                       add=True)
                cs1.wait()
                cs2.wait()

                @pl.loop(0, _C // 16)
                def _(g):
                    rows = lax.iota(jnp.int32, 16) + g * 16
                    lane = jnp.bitwise_and(dbuf[pl.ds(g * 16, 16)], 15) * 8
                    for h in range(4):
                        plsc.store_scatter(exbuf, [rows, lane + h], zero16f)

        plsc.subcore_barrier()

        @pl.loop(0, 78)
        def _(j):
            pltpu.sync_copy(num_sh.at[pl.ds(row0 + j * 8, 8)],
                            num_hbm.at[pl.ds(row0 + j * 8, 8)])

        @pl.loop(0, 5)
        def _(j):
            pltpu.sync_copy(den_sh.at[pl.ds(row0d + j * 8, 8)],
                            den_hbm.at[pl.ds(row0d + j * 8, 8)])

        @pl.when(sid == 15)
        def _():
            pltpu.sync_copy(num_sh.at[pl.ds(row0 + 624, 8)],
                            num_hbm.at[pl.ds(row0 + 624, 8)])
            pltpu.sync_copy(num_sh.at[pl.ds(row0 + 632, 8)],
                            num_hbm.at[pl.ds(row0 + 632, 8)])

    return ek


def _stack_params(P):
    ws, bs, wes, uws, mvecs, gbs, flags = [], [], [], [], [], [], []
    specs = [('enc0', False, True), ('enc1', False, True),
             ('dec0', True, True), ('dec1', True, False)]
    bns = [('bn0_g', 'bn0_b'), ('bn1_g', 'bn1_b'), ('bn2_g', 'bn2_b'), None]
    for (name, wide, four_heads), bn in zip(specs, bns):
        p = P[name]
        wcat = jnp.concatenate([p['Wq'], p['Wk'], p['Wv'], p['Ws']], axis=1)
        if not wide:
            wcat = jnp.concatenate(
                [wcat, jnp.zeros((128, 512), jnp.float32)], axis=0)
        ws.append(wcat)
        bs.append(jnp.concatenate(
            [p['bq'], p['bk'], p['bv'], p['bs']]).reshape(1, 512))
        wes.append(p['We'])
        wb = p['Wb'][:, 0]
        uws.append(jnp.stack(
            [wb[0:128] + wb[256:384], wb[128:256] - wb[256:384]], axis=1))
        mv = jnp.zeros((16,), jnp.float32)
        if four_heads:
            mv = mv.at[0].set(1.0 / math.sqrt(32.0))
        else:
            mv = mv.at[1].set(1.0 / math.sqrt(128.0))
        mvecs.append(mv)
        if bn is None:
            gbs.append(jnp.zeros((2, 128), jnp.float32))
            flags.append(jnp.zeros((1, 128), jnp.float32))
        else:
            gbs.append(jnp.stack([P[bn[0]], P[bn[1]]]))
            flags.append(jnp.ones((1, 128), jnp.float32))
    sels = [jnp.zeros((1, 128), jnp.float32),
            jnp.zeros((1, 128), jnp.float32),
            jnp.zeros((1, 128), jnp.float32).at[0, 0].set(1.0),
            jnp.zeros((1, 128), jnp.float32).at[0, 1].set(1.0)]
    keep_h1 = jnp.array([1.0, 0.0, 0.0, 0.0], jnp.float32)
    return (jnp.stack(ws), jnp.stack(bs), jnp.stack(wes), jnp.stack(uws),
            jnp.stack(mvecs), jnp.stack(gbs), jnp.stack(flags),
            jnp.stack(sels), keep_h1)


def kernel(x, edge_index, edge_attr, params):
    src = edge_index[0]
    dst = edge_index[1]
    xs = _stack_params(params)
    bmat = (jnp.arange(128)[None, :] // 32 == jnp.arange(8)[:, None]
            ).astype(jnp.float32)
    ek = _make_edge_kernel()

    def body(carry, per):
        h, h1s = carry
        wcat, bcat, we, uw, mvec, gb, flag, sel, keep = per
        q, k, v, xr = _proj(h, h1s, sel, wcat, bcat)
        e_all = _emm(edge_attr, we)
        num, denp = ek(q, k, v, e_all, src, dst, mvec)
        den = denp.reshape(640 * 16, 8)[:N_NODES]
        y, stats = _post_combine(num, den, xr, uw, bmat)
        z = _bn_apply(y, stats, gb, flag)
        h1s_new = h1s * (1.0 - keep) + z * keep
        return (z, h1s_new), None

    (out, _), _ = jax.lax.scan(
        body, (x, jnp.zeros((N_NODES, 128), jnp.float32)), xs)
    return out


# speedup vs baseline: 4.9262x; 1.8137x over previous
import dataclasses
import functools
import math

import jax
import jax.numpy as jnp
from jax import lax
from jax.experimental import pallas as pl
from jax.experimental.pallas import tpu as pltpu
from jax.experimental.pallas import tpu_sc as plsc

N_NODES = 10000
N_EDGES = 320000

_ROWT = 1000
_EDGT = 2000

_HIGHEST = jax.lax.Precision.HIGHEST


def _mm(a, b):
    return jax.lax.dot_general(a, b, (((1,), (0,)), ((), ())),
                               preferred_element_type=jnp.float32,
                               precision=_HIGHEST)


def _proj_body(h_ref, h1_ref, sel_ref, w_ref, b_ref,
               q_ref, k_ref, v_ref, s_ref):
    s0 = sel_ref[0:1, 0:1]
    s1 = sel_ref[0:1, 1:2]
    x2 = h_ref[...] * s0 + h1_ref[...] * s1
    xin = jnp.concatenate([h_ref[...], x2], axis=1)
    acc = _mm(xin, w_ref[...]) + b_ref[...]
    q_ref[...] = acc[:, 0:128]
    k_ref[...] = acc[:, 128:256]
    v_ref[...] = acc[:, 256:384]
    s_ref[...] = acc[:, 384:512]


def _proj(h, h1s, sel, wcat, bcat):
    grid = N_NODES // _ROWT
    out = jax.ShapeDtypeStruct((N_NODES, 128), jnp.float32)
    return pl.pallas_call(
        _proj_body,
        grid=(grid,),
        in_specs=[
            pl.BlockSpec((_ROWT, 128), lambda i: (i, 0)),
            pl.BlockSpec((_ROWT, 128), lambda i: (i, 0)),
            pl.BlockSpec((1, 128), lambda i: (0, 0)),
            pl.BlockSpec((256, 512), lambda i: (0, 0)),
            pl.BlockSpec((1, 512), lambda i: (0, 0)),
        ],
        out_specs=[pl.BlockSpec((_ROWT, 128), lambda i: (i, 0))] * 4,
        out_shape=[out, out, out, out],
    )(h, h1s, sel, wcat, bcat)


def _emm_body(a_ref, w_ref, o_ref):
    o_ref[...] = _mm(a_ref[...], w_ref[...])


def _emm(ea, we):
    grid = N_EDGES // _EDGT
    return pl.pallas_call(
        _emm_body,
        grid=(grid,),
        in_specs=[
            pl.BlockSpec((_EDGT, 16), lambda i: (i, 0)),
            pl.BlockSpec((16, 128), lambda i: (0, 0)),
        ],
        out_specs=pl.BlockSpec((_EDGT, 128), lambda i: (i, 0)),
        out_shape=jax.ShapeDtypeStruct((N_EDGES, 128), jnp.float32),
    )(ea, we)


def _post_body(num_ref, den_ref, xr_ref, uw_ref, bmat_ref, y_ref, st_ref):
    den_full = _mm(den_ref[0] + den_ref[1], bmat_ref[...])
    out = (num_ref[0] + num_ref[1]) / (den_full + 1e-16)
    xr = xr_ref[...]
    g = _mm(out, uw_ref[:, 0:1]) + _mm(xr, uw_ref[:, 1:2])
    b = jax.nn.sigmoid(g)
    y = b * xr + (1.0 - b) * out
    y_ref[...] = y

    @pl.when(pl.program_id(0) == 0)
    def _():
        st_ref[...] = jnp.zeros_like(st_ref)

    st_ref[0:1, :] += jnp.sum(y, axis=0, keepdims=True)
    st_ref[1:2, :] += jnp.sum(y * y, axis=0, keepdims=True)


def _post_combine(num, den, xr, uw, bmat):
    grid = N_NODES // _ROWT
    return pl.pallas_call(
        _post_body,
        grid=(grid,),
        in_specs=[
            pl.BlockSpec((2, _ROWT, 128), lambda i: (0, i, 0)),
            pl.BlockSpec((2, _ROWT, 8), lambda i: (0, i, 0)),
            pl.BlockSpec((_ROWT, 128), lambda i: (i, 0)),
            pl.BlockSpec((128, 2), lambda i: (0, 0)),
            pl.BlockSpec((8, 128), lambda i: (0, 0)),
        ],
        out_specs=[
            pl.BlockSpec((_ROWT, 128), lambda i: (i, 0)),
            pl.BlockSpec((2, 128), lambda i: (0, 0)),
        ],
        out_shape=[
            jax.ShapeDtypeStruct((N_NODES, 128), jnp.float32),
            jax.ShapeDtypeStruct((2, 128), jnp.float32),
        ],
    )(num, den, xr, uw, bmat)


def _bn_body(y_ref, st_ref, gb_ref, fl_ref, o_ref):
    inv_n = 1.0 / float(N_NODES)
    m = st_ref[0:1, :] * inv_n
    ex2 = st_ref[1:2, :] * inv_n
    var = ex2 - m * m
    inv = jax.lax.rsqrt(var + 1e-5)
    y = y_ref[...]
    z = (y - m) * inv * gb_ref[0:1, :] + gb_ref[1:2, :]
    z = jnp.maximum(z, 0.0)
    f = fl_ref[0:1, 0:1]
    o_ref[...] = z * f + y * (1.0 - f)


def _bn_apply(y, stats, gb, flag):
    grid = N_NODES // _ROWT
    return pl.pallas_call(
        _bn_body,
        grid=(grid,),
        in_specs=[
            pl.BlockSpec((_ROWT, 128), lambda i: (i, 0)),
            pl.BlockSpec((2, 128), lambda i: (0, 0)),
            pl.BlockSpec((2, 128), lambda i: (0, 0)),
            pl.BlockSpec((1, 128), lambda i: (0, 0)),
        ],
        out_specs=pl.BlockSpec((_ROWT, 128), lambda i: (i, 0)),
        out_shape=jax.ShapeDtypeStruct((N_NODES, 128), jnp.float32),
    )(y, stats, gb, flag)


_C = 64
_NCHUNK = N_EDGES // _C
_CPT = (_NCHUNK + 15) // 16
_RPT = 624


def _sc_params():
    cp = pltpu.CompilerParams()
    if "needs_layout_passes" in pltpu.CompilerParams.__dataclass_fields__:
        cp = dataclasses.replace(cp, needs_layout_passes=False)
    return cp


def _make_edge_kernel():
    mesh = plsc.VectorSubcoreMesh(core_axis_name="c", subcore_axis_name="s")

    @functools.partial(
        pl.kernel,
        compiler_params=_sc_params(),
        out_type=(jax.ShapeDtypeStruct((2, N_NODES, 128), jnp.float32),
                  jax.ShapeDtypeStruct((2, 640, 128), jnp.float32)),
        mesh=mesh,
        scratch_types=[
            pltpu.VMEM((_C, 128), jnp.float32),
            pltpu.VMEM((_C, 128), jnp.float32),
            pltpu.VMEM((_C, 128), jnp.float32),
            pltpu.VMEM((_C,), jnp.int32),
            pltpu.VMEM((_C,), jnp.int32),
            pltpu.VMEM((_C,), jnp.int32),
            pltpu.VMEM((_C, 128), jnp.float32),
            pltpu.VMEM((16,), jnp.float32),
            pltpu.VMEM((8, 128), jnp.float32),
            pltpu.VMEM_SHARED((N_NODES, 128), jnp.float32),
            pltpu.VMEM_SHARED((640, 128), jnp.float32),
            pltpu.SemaphoreType.DMA,
            pltpu.SemaphoreType.DMA,
            pltpu.SemaphoreType.DMA,
        ],
    )
    def ek(q_hbm, k_hbm, v_hbm, e_hbm, src_hbm, dst_hbm, m_hbm,
           num_hbm, den_hbm,
           qbuf, kbuf, ebuf, sbuf, dbuf, dbuf16, exbuf, mbuf, zbuf,
           num_sh, den_sh, sem_i, sem_g, sem_s):
        sid = lax.axis_index("s")
        cid = lax.axis_index("c")
        zero16f = jnp.zeros((16,), jnp.float32)
        pltpu.sync_copy(m_hbm, mbuf)

        @pl.loop(0, 8)
        def _(r):
            for c in range(8):
                zbuf[r, pl.ds(c * 16, 16)] = zero16f

        @pl.loop(0, _C)
        def _(r):
            for c in range(8):
                exbuf[r, pl.ds(c * 16, 16)] = zero16f

        row0 = sid * _RPT
        row0d = sid * 40

        @pl.loop(0, 78)
        def _(j):
            pltpu.sync_copy(zbuf, num_sh.at[pl.ds(row0 + j * 8, 8)])

        @pl.loop(0, 5)
        def _(j):
            pltpu.sync_copy(zbuf, den_sh.at[pl.ds(row0d + j * 8, 8)])

        @pl.when(sid == 15)
        def _():
            pltpu.sync_copy(zbuf, num_sh.at[pl.ds(row0 + 624, 8)])
            pltpu.sync_copy(zbuf, num_sh.at[pl.ds(row0 + 632, 8)])

        plsc.subcore_barrier()
        mv = mbuf[:]
        ma = mv[0]
        mb = mv[1]

        half = _NCHUNK // 2

        @pl.loop(0, _CPT)
        def _(j):
            cidx = cid * half + sid + j * 16

            @pl.when(cidx < cid * half + half)
            def _():
                base = cidx * _C
                ci1 = pltpu.async_copy(src_hbm.at[pl.ds(base, _C)], sbuf,
                                       sem_i)
                ci2 = pltpu.async_copy(dst_hbm.at[pl.ds(base, _C)], dbuf,
                                       sem_i)
                ce = pltpu.async_copy(e_hbm.at[pl.ds(base, _C)], ebuf, sem_g)
                ci1.wait()
                ci2.wait()
                cq = pltpu.async_copy(q_hbm.at[dbuf], qbuf, sem_g)
                ck = pltpu.async_copy(k_hbm.at[sbuf], kbuf, sem_g)
                ce.wait()
                cq.wait()
                ck.wait()

                @pl.loop(0, _C // 16)
                def _(g):
                    iota = lax.iota(jnp.int32, 16)
                    rows = iota + g * 16
                    sl = pl.ds(g * 16, 16)
                    dstv = dbuf[sl]
                    dbuf16[sl] = jnp.right_shift(dstv, 4)
                    lane = jnp.bitwise_and(dstv, 15) * 8
                    ss = []
                    for h in range(4):
                        acc0 = jnp.zeros((16,), jnp.float32)
                        acc1 = jnp.zeros((16,), jnp.float32)
                        for w in range(2):
                            base = h * 32 + 16 * w
                            for t in range(16):
                                col = jnp.bitwise_and(iota + t, 15) + base
                                qv = plsc.load_gather(qbuf, [rows, col])
                                kv = plsc.load_gather(kbuf, [rows, col])
                                ev = plsc.load_gather(ebuf, [rows, col])
                                if t % 2 == 0:
                                    acc0 = acc0 + qv * (kv + ev)
                                else:
                                    acc1 = acc1 + qv * (kv + ev)
                        ss.append(acc0 + acc1)
                    stot = (ss[0] + ss[1]) + (ss[2] + ss[3])
                    for h in range(4):
                        exh = jnp.exp(ss[h] * ma + stot * mb)
                        plsc.store_scatter(exbuf, [rows, lane + h], exh)

                pltpu.async_copy(v_hbm.at[sbuf], qbuf, sem_g).wait()

                @pl.loop(0, _C // 16)
                def _(g):
                    iota = lax.iota(jnp.int32, 16)
                    rows = iota + g * 16
                    sl = pl.ds(g * 16, 16)
                    lane = jnp.bitwise_and(dbuf[sl], 15) * 8
                    for h in range(4):
                        exh = plsc.load_gather(exbuf, [rows, lane + h])
                        for w in range(2):
                            base = h * 32 + 16 * w
                            for t in range(16):
                                col = jnp.bitwise_and(iota + t, 15) + base
                                vv = plsc.load_gather(qbuf, [rows, col])
                                ev = plsc.load_gather(ebuf, [rows, col])
                                plsc.store_scatter(
                                    qbuf, [rows, col], (vv + ev) * exh)

                cs1 = pltpu.async_copy(qbuf, num_sh.at[dbuf], sem_s,
                                       add=True)
                cs2 = pltpu.async_copy(exbuf, den_sh.at[dbuf16], sem_s,
                                       add=True)
                cs1.wait()
                cs2.wait()

                @pl.loop(0, _C // 16)
                def _(g):
                    rows = lax.iota(jnp.int32, 16) + g * 16
                    lane = jnp.bitwise_and(dbuf[pl.ds(g * 16, 16)], 15) * 8
                    for h in range(4):
                        plsc.store_scatter(exbuf, [rows, lane + h], zero16f)

        plsc.subcore_barrier()

        @pl.loop(0, 78)
        def _(j):
            pltpu.sync_copy(num_sh.at[pl.ds(row0 + j * 8, 8)],
                            num_hbm.at[cid, pl.ds(row0 + j * 8, 8)])

        @pl.loop(0, 5)
        def _(j):
            pltpu.sync_copy(den_sh.at[pl.ds(row0d + j * 8, 8)],
                            den_hbm.at[cid, pl.ds(row0d + j * 8, 8)])

        @pl.when(sid == 15)
        def _():
            pltpu.sync_copy(num_sh.at[pl.ds(row0 + 624, 8)],
                            num_hbm.at[cid, pl.ds(row0 + 624, 8)])
            pltpu.sync_copy(num_sh.at[pl.ds(row0 + 632, 8)],
                            num_hbm.at[cid, pl.ds(row0 + 632, 8)])

    return ek


def _stack_params(P):
    ws, bs, wes, uws, mvecs, gbs, flags = [], [], [], [], [], [], []
    specs = [('enc0', False, True), ('enc1', False, True),
             ('dec0', True, True), ('dec1', True, False)]
    bns = [('bn0_g', 'bn0_b'), ('bn1_g', 'bn1_b'), ('bn2_g', 'bn2_b'), None]
    for (name, wide, four_heads), bn in zip(specs, bns):
        p = P[name]
        wcat = jnp.concatenate([p['Wq'], p['Wk'], p['Wv'], p['Ws']], axis=1)
        if not wide:
            wcat = jnp.concatenate(
                [wcat, jnp.zeros((128, 512), jnp.float32)], axis=0)
        ws.append(wcat)
        bs.append(jnp.concatenate(
            [p['bq'], p['bk'], p['bv'], p['bs']]).reshape(1, 512))
        wes.append(p['We'])
        wb = p['Wb'][:, 0]
        uws.append(jnp.stack(
            [wb[0:128] + wb[256:384], wb[128:256] - wb[256:384]], axis=1))
        mv = jnp.zeros((16,), jnp.float32)
        if four_heads:
            mv = mv.at[0].set(1.0 / math.sqrt(32.0))
        else:
            mv = mv.at[1].set(1.0 / math.sqrt(128.0))
        mvecs.append(mv)
        if bn is None:
            gbs.append(jnp.zeros((2, 128), jnp.float32))
            flags.append(jnp.zeros((1, 128), jnp.float32))
        else:
            gbs.append(jnp.stack([P[bn[0]], P[bn[1]]]))
            flags.append(jnp.ones((1, 128), jnp.float32))
    sels = [jnp.zeros((1, 128), jnp.float32),
            jnp.zeros((1, 128), jnp.float32),
            jnp.zeros((1, 128), jnp.float32).at[0, 0].set(1.0),
            jnp.zeros((1, 128), jnp.float32).at[0, 1].set(1.0)]
    keep_h1 = jnp.array([1.0, 0.0, 0.0, 0.0], jnp.float32)
    return (jnp.stack(ws), jnp.stack(bs), jnp.stack(wes), jnp.stack(uws),
            jnp.stack(mvecs), jnp.stack(gbs), jnp.stack(flags),
            jnp.stack(sels), keep_h1)


def kernel(x, edge_index, edge_attr, params):
    src = edge_index[0]
    dst = edge_index[1]
    xs = _stack_params(params)
    bmat = (jnp.arange(128)[None, :] // 32 == jnp.arange(8)[:, None]
            ).astype(jnp.float32)
    ek = _make_edge_kernel()

    def body(carry, per):
        h, h1s = carry
        wcat, bcat, we, uw, mvec, gb, flag, sel, keep = per
        q, k, v, xr = _proj(h, h1s, sel, wcat, bcat)
        e_all = _emm(edge_attr, we)
        num, denp = ek(q, k, v, e_all, src, dst, mvec)
        den = denp.reshape(2, 640 * 16, 8)[:, :N_NODES]
        y, stats = _post_combine(num, den, xr, uw, bmat)
        z = _bn_apply(y, stats, gb, flag)
        h1s_new = h1s * (1.0 - keep) + z * keep
        return (z, h1s_new), None

    (out, _), _ = jax.lax.scan(
        body, (x, jnp.zeros((N_NODES, 128), jnp.float32)), xs)
    return out
